# baseline (speedup 1.0000x reference)
import jax
import jax.numpy as jnp
from jax.experimental import pallas as pl
from jax.experimental.pallas import tpu as pltpu

QSIZE = 65536
DIM = 128
B = 1024
NCHUNK = 16
CHUNK = QSIZE // NCHUNK


def _body(ptr_ref, queue_ref, key_ref, out_ref, copy_sems, key_sem):
    copies = [
        pltpu.make_async_copy(
            queue_ref.at[pl.ds(i * CHUNK, CHUNK)],
            out_ref.at[pl.ds(i * CHUNK, CHUNK)],
            copy_sems.at[i],
        )
        for i in range(NCHUNK)
    ]
    for c in copies:
        c.start()
    for c in copies:
        c.wait()
    p = ptr_ref[0]
    kc = pltpu.make_async_copy(key_ref, out_ref.at[pl.ds(p, B)], key_sem)
    kc.start()
    kc.wait()


def kernel(queue, key_batch, queue_ptr):
    ptr = jnp.asarray(queue_ptr, jnp.int32).reshape((1,))
    return pl.pallas_call(
        _body,
        out_shape=jax.ShapeDtypeStruct((QSIZE, DIM), jnp.float32),
        in_specs=[
            pl.BlockSpec(memory_space=pltpu.SMEM),
            pl.BlockSpec(memory_space=pl.ANY),
            pl.BlockSpec(memory_space=pl.ANY),
        ],
        out_specs=pl.BlockSpec(memory_space=pl.ANY),
        scratch_shapes=[
            pltpu.SemaphoreType.DMA((NCHUNK,)),
            pltpu.SemaphoreType.DMA,
        ],
    )(ptr, queue, key_batch)


# speedup vs baseline: 40.3785x; 40.3785x over previous
import jax
import jax.numpy as jnp
from jax.experimental import pallas as pl
from jax.experimental.pallas import tpu as pltpu

QSIZE = 65536
DIM = 128
B = 1024
NBLK = 16
BLK = QSIZE // NBLK


def _body(ptr_ref, q_ref, k_ref, o_ref):
    i = pl.program_id(0)
    p = ptr_ref[0]
    o_ref[...] = q_ref[...]

    @pl.when(i == p // BLK)
    def _():
        o_ref[pl.ds(p % BLK, B), :] = k_ref[...]


def kernel(queue, key_batch, queue_ptr):
    ptr = jnp.asarray(queue_ptr, jnp.int32).reshape((1,))
    return pl.pallas_call(
        _body,
        grid=(NBLK,),
        out_shape=jax.ShapeDtypeStruct((QSIZE, DIM), jnp.float32),
        in_specs=[
            pl.BlockSpec(memory_space=pltpu.SMEM),
            pl.BlockSpec((BLK, DIM), lambda i: (i, 0)),
            pl.BlockSpec((B, DIM), lambda i: (0, 0)),
        ],
        out_specs=pl.BlockSpec((BLK, DIM), lambda i: (i, 0)),
    )(ptr, queue, key_batch)


# speedup vs baseline: 44.0191x; 1.0902x over previous
import jax
import jax.numpy as jnp
from jax.experimental import pallas as pl
from jax.experimental.pallas import tpu as pltpu

QSIZE = 65536
DIM = 128
B = 1024
NBLK = 8
BLK = QSIZE // NBLK


def _body(ptr_ref, q_ref, k_ref, o_ref):
    i = pl.program_id(0)
    p = ptr_ref[0]
    o_ref[...] = q_ref[...]

    @pl.when(i == p // BLK)
    def _():
        o_ref[pl.ds(p % BLK, B), :] = k_ref[...]


def kernel(queue, key_batch, queue_ptr):
    ptr = jnp.asarray(queue_ptr, jnp.int32).reshape((1,))
    return pl.pallas_call(
        _body,
        grid=(NBLK,),
        out_shape=jax.ShapeDtypeStruct((QSIZE, DIM), jnp.float32),
        in_specs=[
            pl.BlockSpec(memory_space=pltpu.SMEM),
            pl.BlockSpec((BLK, DIM), lambda i: (i, 0)),
            pl.BlockSpec((B, DIM), lambda i: (0, 0)),
        ],
        out_specs=pl.BlockSpec((BLK, DIM), lambda i: (i, 0)),
    )(ptr, queue, key_batch)


# speedup vs baseline: 47.0232x; 1.0682x over previous
import jax
import jax.numpy as jnp
from jax.experimental import pallas as pl
from jax.experimental.pallas import tpu as pltpu

QSIZE = 65536
DIM = 128
B = 1024
NBLK = 4
BLK = QSIZE // NBLK


def _body(ptr_ref, q_ref, k_ref, o_ref):
    i = pl.program_id(0)
    p = ptr_ref[0]
    o_ref[...] = q_ref[...]

    @pl.when(i == p // BLK)
    def _():
        o_ref[pl.ds(p % BLK, B), :] = k_ref[...]


def kernel(queue, key_batch, queue_ptr):
    ptr = jnp.asarray(queue_ptr, jnp.int32).reshape((1,))
    return pl.pallas_call(
        _body,
        grid=(NBLK,),
        out_shape=jax.ShapeDtypeStruct((QSIZE, DIM), jnp.float32),
        in_specs=[
            pl.BlockSpec(memory_space=pltpu.SMEM),
            pl.BlockSpec((BLK, DIM), lambda i: (i, 0)),
            pl.BlockSpec((B, DIM), lambda i: (0, 0)),
        ],
        out_specs=pl.BlockSpec((BLK, DIM), lambda i: (i, 0)),
    )(ptr, queue, key_batch)
